# baseline (speedup 1.0000x reference)
import jax
import jax.numpy as jnp
from jax.experimental import pallas as pl
from jax.experimental.pallas import tpu as pltpu

_N = 1024
_H = 16
_K = 8
_R_MAX = 5.0
_EPS = 1e-8
_AVG_NB = float(_N - 1)
_T = 2


def _mace_body(pos_ref, posT_ref, nf_ref, se_ref, wrbf_ref, wupd_ref,
               wvec_ref, gf_ref, wglob_ref, fs_ref, out_ref,
               h_scr, vec_scr, rbf_scr, len_scr):
    t = pl.program_id(0)

    pos = pos_ref[:]
    posT = posT_ref[:]

    @pl.when(t == 0)
    def _init():
        onehot = (jax.lax.broadcasted_iota(jnp.int32, (_N, 8), 1)
                  == nf_ref[:]).astype(jnp.float32)
        h_scr[:] = jnp.dot(onehot, se_ref[:],
                           preferred_element_type=jnp.float32)
        vec_scr[:] = jnp.zeros((_N, 3), jnp.float32)

        d2 = jnp.full((_N, _N), _EPS, jnp.float32)
        for j in range(3):
            dj = pos[:, j:j + 1] - posT[j:j + 1, :]
            d2 = d2 + dj * dj
        length = jnp.sqrt(d2)
        len_scr[:] = length

        cut = 0.5 * (jnp.cos(jnp.pi * jnp.clip(length * (1.0 / _R_MAX),
                                               0.0, 1.0)) + 1.0)
        rr = jax.lax.broadcasted_iota(jnp.int32, (_N, _N), 0)
        ss = jax.lax.broadcasted_iota(jnp.int32, (_N, _N), 1)
        cut = jnp.where(rr == ss, 0.0, cut)
        for k in range(_K):
            mu = _R_MAX * k / (_K - 1)
            rbf_scr[k] = (jnp.exp(-((length - mu) ** 2))
                          * cut).astype(jnp.bfloat16)

    h = h_scr[:]
    length = len_scr[:]

    wrbf = wrbf_ref[0]
    wvec = wvec_ref[0]
    m_kc = wrbf * wvec
    qT = jax.lax.dot_general(m_kc, h, (((1,), (1,)), ((), ())),
                             preferred_element_type=jnp.float32)

    h_b = h.astype(jnp.bfloat16)
    agg = jnp.zeros((_N, _H), jnp.float32)
    tacc = jnp.zeros((_N, _N), jnp.float32)
    for k in range(_K):
        rbf = rbf_scr[k]
        agg = agg + jnp.dot(rbf, h_b,
                            preferred_element_type=jnp.float32) * wrbf[k, :]
        tacc = tacc + rbf.astype(jnp.float32) * qT[k:k + 1, :]

    agg = agg * (1.0 / _AVG_NB)
    h_scr[:] = jnp.tanh(jnp.dot(agg, wupd_ref[0],
                                preferred_element_type=jnp.float32)) + h

    tmat = tacc / length
    rowsum = jnp.sum(tmat, axis=1, keepdims=True)
    tp = jnp.dot(tmat, pos, preferred_element_type=jnp.float32)
    vec_scr[:] = vec_scr[:] + (rowsum * pos - tp) * (1.0 / _AVG_NB)

    @pl.when(t == _T - 1)
    def _fin():
        gate = 1.0 + jnp.tanh(jnp.sum(gf_ref[:] * wglob_ref[:]))
        out_ref[:] = (vec_scr[:] * gate - pos) * fs_ref[0, 0]


def kernel(positions, node_features, global_features, species_embed,
           W_rbf, W_upd, w_vec, w_glob, final_scaling):
    pos = positions.astype(jnp.float32)
    posT = pos.T
    nf = node_features.astype(jnp.int32).reshape(_N, 1)
    se = jnp.zeros((8, _H), jnp.float32).at[:species_embed.shape[0]].set(
        species_embed.astype(jnp.float32))
    wrbf = W_rbf.astype(jnp.float32)
    wupd = W_upd.astype(jnp.float32)
    wvec = w_vec.astype(jnp.float32).reshape(_T, 1, _H)
    gf = global_features.astype(jnp.float32).reshape(1, -1)
    wglob = w_glob.astype(jnp.float32).reshape(1, -1)
    fs = final_scaling.astype(jnp.float32).reshape(1, 1)

    grid = (_T,)
    out = pl.pallas_call(
        _mace_body,
        grid=grid,
        in_specs=[
            pl.BlockSpec((_N, 3), lambda t: (0, 0)),
            pl.BlockSpec((3, _N), lambda t: (0, 0)),
            pl.BlockSpec((_N, 1), lambda t: (0, 0)),
            pl.BlockSpec((8, _H), lambda t: (0, 0)),
            pl.BlockSpec((1, _K, _H), lambda t: (t, 0, 0)),
            pl.BlockSpec((1, _H, _H), lambda t: (t, 0, 0)),
            pl.BlockSpec((1, 1, _H), lambda t: (t, 0, 0)),
            pl.BlockSpec((1, gf.shape[1]), lambda t: (0, 0)),
            pl.BlockSpec((1, wglob.shape[1]), lambda t: (0, 0)),
            pl.BlockSpec((1, 1), lambda t: (0, 0)),
        ],
        out_specs=pl.BlockSpec((_N, 3), lambda t: (0, 0)),
        out_shape=jax.ShapeDtypeStruct((_N, 3), jnp.float32),
        scratch_shapes=[
            pltpu.VMEM((_N, _H), jnp.float32),
            pltpu.VMEM((_N, 3), jnp.float32),
            pltpu.VMEM((_K, _N, _N), jnp.bfloat16),
            pltpu.VMEM((_N, _N), jnp.float32),
        ],
    )(pos, posT, nf, se, wrbf, wupd, wvec, gf, wglob, fs)
    return out


# speedup vs baseline: 2.0770x; 2.0770x over previous
import jax
import jax.numpy as jnp
from jax.experimental import pallas as pl
from jax.experimental.pallas import tpu as pltpu

_N = 1024
_H = 16
_K = 8
_R_MAX = 5.0
_EPS = 1e-8
_AVG_NB = float(_N - 1)
_T = 2


_S0 = 3.1415925994720157
_S1 = -5.1677080818450705
_S2 = 2.5500509887600358
_S3 = -0.5981614666896089
_S4 = 0.07744687538918765
_LOG2E = 1.4426950408889634


def _mace_body(pos_ref, posT_ref, nf_ref, se_ref, wrbf_ref, wupd_ref,
               wvec_ref, gf_ref, wglob_ref, fs_ref, out_ref,
               h_scr, vec_scr):
    t = pl.program_id(0)

    pos = pos_ref[:]
    posT = posT_ref[:]

    @pl.when(t == 0)
    def _init():
        onehot = (jax.lax.broadcasted_iota(jnp.int32, (_N, 8), 1)
                  == nf_ref[:]).astype(jnp.float32)
        h_scr[:] = jnp.dot(onehot, se_ref[:],
                           preferred_element_type=jnp.float32)
        vec_scr[:] = jnp.zeros((_N, 3), jnp.float32)

    h = h_scr[:]

    gram = jnp.dot(pos, posT, preferred_element_type=jnp.float32)
    sq_c = jnp.sum(pos * pos, axis=1, keepdims=True)
    sq_r = jnp.sum(posT * posT, axis=0, keepdims=True)
    d2 = jnp.maximum((sq_c - 2.0 * gram) + sq_r, 0.0) + _EPS
    length = jnp.sqrt(d2)
    lc = jnp.minimum(length, _R_MAX)

    y = lc * (1.0 / _R_MAX) - 0.5
    z = y * y
    sin_pi_y = y * (_S0 + z * (_S1 + z * (_S2 + z * (_S3 + z * _S4))))
    cut = 0.5 - 0.5 * sin_pi_y
    rr = jax.lax.broadcasted_iota(jnp.int32, (_N, _N), 0)
    ss = jax.lax.broadcasted_iota(jnp.int32, (_N, _N), 1)
    cut = jnp.where(rr == ss, 0.0, cut)

    base = jnp.exp2(-(lc * lc) * _LOG2E) * cut

    wrbf = wrbf_ref[0]
    wvec = wvec_ref[0]
    m_kc = wrbf * wvec
    qT = jax.lax.dot_general(m_kc, h, (((1,), (1,)), ((), ())),
                             preferred_element_type=jnp.float32)

    agg = jnp.zeros((_N, _H), jnp.float32)
    tacc = jnp.zeros((_N, _N), jnp.float32)
    for k in range(_K):
        mu = _R_MAX * k / (_K - 1)
        rbf = base * jnp.exp2((2.0 * mu * _LOG2E) * lc
                              + (-mu * mu * _LOG2E))
        agg = agg + jnp.dot(rbf, h,
                            preferred_element_type=jnp.float32) * wrbf[k, :]
        tacc = tacc + rbf * qT[k:k + 1, :]

    agg = agg * (1.0 / _AVG_NB)
    h_scr[:] = jnp.tanh(jnp.dot(agg, wupd_ref[0],
                                preferred_element_type=jnp.float32)) + h

    tmat = tacc / length
    rowsum = jnp.sum(tmat, axis=1, keepdims=True)
    tp = jnp.dot(tmat, pos, preferred_element_type=jnp.float32)
    vec_scr[:] = vec_scr[:] + (rowsum * pos - tp) * (1.0 / _AVG_NB)

    @pl.when(t == _T - 1)
    def _fin():
        gate = 1.0 + jnp.tanh(jnp.sum(gf_ref[:] * wglob_ref[:]))
        out_ref[:] = (vec_scr[:] * gate - pos) * fs_ref[0, 0]


def kernel(positions, node_features, global_features, species_embed,
           W_rbf, W_upd, w_vec, w_glob, final_scaling):
    pos = positions.astype(jnp.float32)
    posT = pos.T
    nf = node_features.astype(jnp.int32).reshape(_N, 1)
    se = jnp.zeros((8, _H), jnp.float32).at[:species_embed.shape[0]].set(
        species_embed.astype(jnp.float32))
    wrbf = W_rbf.astype(jnp.float32)
    wupd = W_upd.astype(jnp.float32)
    wvec = w_vec.astype(jnp.float32).reshape(_T, 1, _H)
    gf = global_features.astype(jnp.float32).reshape(1, -1)
    wglob = w_glob.astype(jnp.float32).reshape(1, -1)
    fs = final_scaling.astype(jnp.float32).reshape(1, 1)

    grid = (_T,)
    out = pl.pallas_call(
        _mace_body,
        grid=grid,
        in_specs=[
            pl.BlockSpec((_N, 3), lambda t: (0, 0)),
            pl.BlockSpec((3, _N), lambda t: (0, 0)),
            pl.BlockSpec((_N, 1), lambda t: (0, 0)),
            pl.BlockSpec((8, _H), lambda t: (0, 0)),
            pl.BlockSpec((1, _K, _H), lambda t: (t, 0, 0)),
            pl.BlockSpec((1, _H, _H), lambda t: (t, 0, 0)),
            pl.BlockSpec((1, 1, _H), lambda t: (t, 0, 0)),
            pl.BlockSpec((1, gf.shape[1]), lambda t: (0, 0)),
            pl.BlockSpec((1, wglob.shape[1]), lambda t: (0, 0)),
            pl.BlockSpec((1, 1), lambda t: (0, 0)),
        ],
        out_specs=pl.BlockSpec((_N, 3), lambda t: (0, 0)),
        out_shape=jax.ShapeDtypeStruct((_N, 3), jnp.float32),
        scratch_shapes=[
            pltpu.VMEM((_N, _H), jnp.float32),
            pltpu.VMEM((_N, 3), jnp.float32),
        ],
    )(pos, posT, nf, se, wrbf, wupd, wvec, gf, wglob, fs)
    return out


# speedup vs baseline: 3.4310x; 1.6519x over previous
import math

import jax
import jax.numpy as jnp
from jax.experimental import pallas as pl

_N = 1024
_H = 16
_K = 8
_R_MAX = 5.0
_EPS = 1e-8
_AVG_NB = float(_N - 1)
_T = 2

_S0 = 3.1415925994720157
_S1 = -5.1677080818450705
_S2 = 2.5500509887600358
_S3 = -0.5981614666896089
_S4 = 0.07744687538918765
_LOG2E = 1.4426950408889634
_DMU = _R_MAX / (_K - 1)
_RHO = [float(math.exp(-(_DMU * k) ** 2)) for k in range(_K)]


def _mace_body(pos_ref, posT_ref, nf_ref, se_ref, wrbf_ref, wupd_ref,
               wvec_ref, gf_ref, wglob_ref, fs_ref, out_ref):
    pos = pos_ref[:]
    posT = posT_ref[:]

    onehot = (jax.lax.broadcasted_iota(jnp.int32, (_N, 8), 1)
              == nf_ref[:]).astype(jnp.float32)
    h = jnp.dot(onehot, se_ref[:], preferred_element_type=jnp.float32)

    gram = jnp.dot(pos, posT, preferred_element_type=jnp.float32)
    sq_c = jnp.sum(pos * pos, axis=1, keepdims=True)
    sq_r = jnp.sum(posT * posT, axis=0, keepdims=True)
    d2 = jnp.maximum((sq_c - 2.0 * gram) + sq_r, 0.0) + _EPS
    length = jnp.sqrt(d2)
    lc = jnp.minimum(length, _R_MAX)

    y = lc * (1.0 / _R_MAX) - 0.5
    z = y * y
    sin_pi_y = y * (_S0 + z * (_S1 + z * (_S2 + z * (_S3 + z * _S4))))
    cut = 0.5 - 0.5 * sin_pi_y
    rr = jax.lax.broadcasted_iota(jnp.int32, (_N, _N), 0)
    ss = jax.lax.broadcasted_iota(jnp.int32, (_N, _N), 1)
    cut = jnp.where(rr == ss, 0.0, cut)

    base = jnp.exp2(-(lc * lc) * _LOG2E) * cut
    u = jnp.exp2((2.0 * _DMU * _LOG2E) * lc)
    inv_len = 1.0 / length

    vec = jnp.zeros((_N, 3), jnp.float32)
    for t in range(_T):
        wrbf = wrbf_ref[t]
        wvec = wvec_ref[t]
        m_kc = wrbf * wvec
        qT = jax.lax.dot_general(m_kc, h, (((1,), (1,)), ((), ())),
                                 preferred_element_type=jnp.float32)

        agg = jnp.zeros((_N, _H), jnp.float32)
        tacc = jnp.zeros((_N, _N), jnp.float32)
        rbf = base
        for k in range(_K):
            agg = agg + jnp.dot(rbf, h,
                                preferred_element_type=jnp.float32) * (
                                    wrbf[k, :] * _RHO[k])
            tacc = tacc + rbf * (qT[k:k + 1, :] * _RHO[k])
            if k + 1 < _K:
                rbf = rbf * u

        agg = agg * (1.0 / _AVG_NB)
        h = jnp.tanh(jnp.dot(agg, wupd_ref[t],
                             preferred_element_type=jnp.float32)) + h

        tmat = tacc * inv_len
        rowsum = jnp.sum(tmat, axis=1, keepdims=True)
        tp = jnp.dot(tmat, pos, preferred_element_type=jnp.float32)
        vec = vec + (rowsum * pos - tp) * (1.0 / _AVG_NB)

    gate = 1.0 + jnp.tanh(jnp.sum(gf_ref[:] * wglob_ref[:]))
    out_ref[:] = (vec * gate - pos) * fs_ref[0, 0]


def kernel(positions, node_features, global_features, species_embed,
           W_rbf, W_upd, w_vec, w_glob, final_scaling):
    pos = positions.astype(jnp.float32)
    posT = pos.T
    nf = node_features.astype(jnp.int32).reshape(_N, 1)
    se = jnp.zeros((8, _H), jnp.float32).at[:species_embed.shape[0]].set(
        species_embed.astype(jnp.float32))
    wrbf = W_rbf.astype(jnp.float32)
    wupd = W_upd.astype(jnp.float32)
    wvec = w_vec.astype(jnp.float32).reshape(_T, 1, _H)
    gf = global_features.astype(jnp.float32).reshape(1, -1)
    wglob = w_glob.astype(jnp.float32).reshape(1, -1)
    fs = final_scaling.astype(jnp.float32).reshape(1, 1)

    out = pl.pallas_call(
        _mace_body,
        out_shape=jax.ShapeDtypeStruct((_N, 3), jnp.float32),
    )(pos, posT, nf, se, wrbf, wupd, wvec, gf, wglob, fs)
    return out
